# baseline (speedup 1.0000x reference)
import jax
import jax.numpy as jnp
from jax.experimental import pallas as pl

_GPB = 4
_BF = jnp.bfloat16


def _dot(a, b):
    return jnp.dot(a, b, preferred_element_type=jnp.float32)


def _cheb_layer(L, x, Wc, b):
    t1 = _dot(L, x).astype(_BF)
    t2 = (2.0 * _dot(L, t1) - x.astype(jnp.float32)).astype(_BF)
    cat = jnp.concatenate([x, t1, t2], axis=1)
    return _dot(cat, Wc) + b


def _net_kernel(x_ref, a_ref, w1_ref, b1_ref, w2_ref, b2_ref, w3_ref, b3_ref,
                o_ref):
    for g in range(_GPB):
        A = a_ref[g]
        deg = jnp.sum(A, axis=-1)
        dinv = jnp.where(deg > 0.0,
                         jax.lax.rsqrt(jnp.where(deg > 0.0, deg, 1.0)), 0.0)
        L = (-(A * dinv[:, None] * dinv[None, :])).astype(_BF)

        x = x_ref[g]
        h = jax.nn.relu(_cheb_layer(L, x, w1_ref[...], b1_ref[...])).astype(_BF)
        h = jax.nn.relu(_cheb_layer(L, h, w2_ref[...], b2_ref[...])).astype(_BF)
        o_ref[g] = _cheb_layer(L, h, w3_ref[...], b3_ref[...])


def kernel(X, A, W1, b1, W2, b2, W3, b3):
    B, S, T, E = X.shape
    d_in = T * E
    d_out = W3.shape[-1]
    x = X.reshape(B, S, d_in).astype(_BF)
    Wc1 = W1.reshape(-1, W1.shape[-1]).astype(_BF)
    Wc2 = W2.reshape(-1, W2.shape[-1]).astype(_BF)
    Wc3 = W3.reshape(-1, W3.shape[-1]).astype(_BF)

    def batch_spec(shape):
        return pl.BlockSpec((_GPB,) + shape, lambda b: (b, 0, 0))

    def full_spec(arr):
        return pl.BlockSpec(arr.shape, lambda b: (0,) * arr.ndim)

    return pl.pallas_call(
        _net_kernel,
        grid=(B // _GPB,),
        in_specs=[
            batch_spec((S, d_in)),
            batch_spec((S, S)),
            full_spec(Wc1), full_spec(b1),
            full_spec(Wc2), full_spec(b2),
            full_spec(Wc3), full_spec(b3),
        ],
        out_specs=batch_spec((S, d_out)),
        out_shape=jax.ShapeDtypeStruct((B, S, d_out), jnp.float32),
    )(x, A, Wc1, b1, Wc2, b2, Wc3, b3)


# speedup vs baseline: 1.0411x; 1.0411x over previous
import jax
import jax.numpy as jnp
from jax.experimental import pallas as pl

_GPB = 4
_BF = jnp.bfloat16


def _dot(a, b):
    return jnp.dot(a, b, preferred_element_type=jnp.float32)


def _cheb_layer(L, P, x, Wc, b):
    t1 = _dot(L, x).astype(_BF)
    t2 = _dot(P, x).astype(_BF)
    cat = jnp.concatenate([x, t1, t2], axis=1)
    return _dot(cat, Wc) + b


def _net_kernel(x_ref, a_ref, w1_ref, b1_ref, w2_ref, b2_ref, w3_ref, b3_ref,
                o_ref):
    S = a_ref.shape[-1]
    rows = jax.lax.broadcasted_iota(jnp.int32, (S, S), 0)
    cols = jax.lax.broadcasted_iota(jnp.int32, (S, S), 1)
    eye = (rows == cols).astype(jnp.float32)
    for g in range(_GPB):
        A = a_ref[g]
        deg = jnp.sum(A, axis=-1)
        dinv = jnp.where(deg > 0.0,
                         jax.lax.rsqrt(jnp.where(deg > 0.0, deg, 1.0)), 0.0)
        L = (-(A * dinv[:, None] * dinv[None, :])).astype(_BF)
        P = (2.0 * _dot(L, L) - eye).astype(_BF)

        x = x_ref[g]
        h = jax.nn.relu(
            _cheb_layer(L, P, x, w1_ref[...], b1_ref[...])).astype(_BF)
        h = jax.nn.relu(
            _cheb_layer(L, P, h, w2_ref[...], b2_ref[...])).astype(_BF)
        o_ref[g] = _cheb_layer(L, P, h, w3_ref[...], b3_ref[...])


def kernel(X, A, W1, b1, W2, b2, W3, b3):
    B, S, T, E = X.shape
    d_in = T * E
    d_out = W3.shape[-1]
    x = X.reshape(B, S, d_in).astype(_BF)
    Wc1 = W1.reshape(-1, W1.shape[-1]).astype(_BF)
    Wc2 = W2.reshape(-1, W2.shape[-1]).astype(_BF)
    Wc3 = W3.reshape(-1, W3.shape[-1]).astype(_BF)

    def batch_spec(shape):
        return pl.BlockSpec((_GPB,) + shape, lambda b: (b, 0, 0))

    def full_spec(arr):
        return pl.BlockSpec(arr.shape, lambda b: (0,) * arr.ndim)

    return pl.pallas_call(
        _net_kernel,
        grid=(B // _GPB,),
        in_specs=[
            batch_spec((S, d_in)),
            batch_spec((S, S)),
            full_spec(Wc1), full_spec(b1),
            full_spec(Wc2), full_spec(b2),
            full_spec(Wc3), full_spec(b3),
        ],
        out_specs=batch_spec((S, d_out)),
        out_shape=jax.ShapeDtypeStruct((B, S, d_out), jnp.float32),
    )(x, A, Wc1, b1, Wc2, b2, Wc3, b3)


# speedup vs baseline: 1.1013x; 1.0578x over previous
import jax
import jax.numpy as jnp
from jax.experimental import pallas as pl

_GPB = 4
_BF = jnp.bfloat16


def _dot(a, b):
    return jnp.dot(a, b, preferred_element_type=jnp.float32)


def _cheb_layer(L, P, x, Wc, b):
    t1 = _dot(L, x).astype(_BF)
    t2 = _dot(P, x).astype(_BF)
    cat = jnp.concatenate([x, t1, t2], axis=1)
    return _dot(cat, Wc) + b


def _net_kernel(x_ref, a_ref, w1_ref, b1_ref, w2_ref, b2_ref, w3_ref, b3_ref,
                o_ref):
    S = a_ref.shape[-1]
    rows = jax.lax.broadcasted_iota(jnp.int32, (S, S), 0)
    cols = jax.lax.broadcasted_iota(jnp.int32, (S, S), 1)
    eye = (rows == cols).astype(jnp.float32)
    for g in range(_GPB):
        A = a_ref[g]
        deg = jnp.sum(A, axis=-1)
        dinv = jnp.where(deg > 0.0,
                         jax.lax.rsqrt(jnp.where(deg > 0.0, deg, 1.0)), 0.0)
        L = (-(A * dinv[:, None] * dinv[None, :])).astype(_BF)
        P = (2.0 * _dot(L, L) - eye).astype(_BF)

        x = x_ref[g].astype(_BF)
        h = jax.nn.relu(
            _cheb_layer(L, P, x, w1_ref[...].astype(_BF), b1_ref[...])
        ).astype(_BF)
        h = jax.nn.relu(
            _cheb_layer(L, P, h, w2_ref[...].astype(_BF), b2_ref[...])
        ).astype(_BF)
        o_ref[g] = _cheb_layer(L, P, h, w3_ref[...].astype(_BF), b3_ref[...])


def kernel(X, A, W1, b1, W2, b2, W3, b3):
    B, S, T, E = X.shape
    d_in = T * E
    d_out = W3.shape[-1]
    x = X.reshape(B, S, d_in)
    Wc1 = W1.reshape(-1, W1.shape[-1])
    Wc2 = W2.reshape(-1, W2.shape[-1])
    Wc3 = W3.reshape(-1, W3.shape[-1])

    def batch_spec(shape):
        return pl.BlockSpec((_GPB,) + shape, lambda b: (b, 0, 0))

    def full_spec(arr):
        return pl.BlockSpec(arr.shape, lambda b: (0,) * arr.ndim)

    return pl.pallas_call(
        _net_kernel,
        grid=(B // _GPB,),
        in_specs=[
            batch_spec((S, d_in)),
            batch_spec((S, S)),
            full_spec(Wc1), full_spec(b1),
            full_spec(Wc2), full_spec(b2),
            full_spec(Wc3), full_spec(b3),
        ],
        out_specs=batch_spec((S, d_out)),
        out_shape=jax.ShapeDtypeStruct((B, S, d_out), jnp.float32),
    )(x, A, Wc1, b1, Wc2, b2, Wc3, b3)
